# baseline (speedup 1.0000x reference)
import functools

import jax
import jax.numpy as jnp
from jax.experimental import pallas as pl
from jax.experimental.pallas import tpu as pltpu

S = 2048
E = 8
M = 768
DFF = 3072
CUMCHUNK = 512
FFCHUNK = 384
NFF = DFF // FFCHUNK


def _fused_kernel(
    x_ref, wg_ref, bg_ref, w1_ref, b1_ref, w2_ref, b2_ref, out_ref,
    coeff_ref, xb_ref,
):
    step = pl.program_id(0)

    @pl.when(step == 0)
    def _gating():
        xb_ref[...] = x_ref[...].astype(jnp.bfloat16)

        logits = jnp.dot(x_ref[...], wg_ref[...], preferred_element_type=jnp.float32)
        logits = logits + bg_ref[...]
        lt = jnp.transpose(logits)
        row = jax.lax.broadcasted_iota(jnp.int32, (E, S), 0)

        max1 = jnp.max(lt, axis=0, keepdims=True)
        idx1 = jnp.min(jnp.where(lt == max1, row, E), axis=0, keepdims=True)
        m1 = row == idx1

        neg_inf = jnp.float32(-jnp.inf)
        le1 = jnp.where(m1, neg_inf, lt)
        max2 = jnp.max(le1, axis=0, keepdims=True)
        idx2 = jnp.min(jnp.where(le1 == max2, row, E), axis=0, keepdims=True)
        m2 = row == idx2
        m1f = m1.astype(jnp.float32)
        m2f = m2.astype(jnp.float32)

        expx = jnp.exp(lt - max1)
        denom = jnp.sum(expx, axis=0, keepdims=True)
        g1 = jnp.sum(jnp.where(m1, expx, 0.0), axis=0, keepdims=True) / denom
        g2 = jnp.sum(jnp.where(m2, expx, 0.0), axis=0, keepdims=True) / denom

        cs2 = jnp.zeros((E, S), jnp.float32)
        for c in range(S // CUMCHUNK):
            t0 = c * CUMCHUNK
            rr = jax.lax.broadcasted_iota(jnp.int32, (CUMCHUNK, S), 0)
            cc = jax.lax.broadcasted_iota(jnp.int32, (CUMCHUNK, S), 1)
            ut = (rr + t0 <= cc).astype(jnp.float32)
            chunk = m2f[:, t0 : t0 + CUMCHUNK]
            cs2 = cs2 + jnp.dot(chunk, ut, preferred_element_type=jnp.float32)

        count1 = jnp.sum(m1f, axis=1, keepdims=True)
        loc2 = (
            jnp.sum(cs2 * m2f, axis=0, keepdims=True)
            - 1.0
            + jnp.sum(count1 * m2f, axis=0, keepdims=True)
        )
        valid2 = (loc2 < jnp.float32(S)).astype(jnp.float32)

        den = jnp.maximum(g1 + g2, jnp.float32(jnp.finfo(jnp.float32).eps))
        coeff_ref[...] = jnp.transpose((g1 + g2 * valid2) / den)

    @pl.when(step > 0)
    def _ffn():
        w1c = w1_ref[...].astype(jnp.bfloat16)
        w2c = w2_ref[...].astype(jnp.bfloat16)
        h = jnp.dot(xb_ref[...], w1c, preferred_element_type=jnp.float32)
        h = jnp.maximum(h + b1_ref[...], 0.0).astype(jnp.bfloat16)
        y = jnp.dot(h, w2c, preferred_element_type=jnp.float32)

        @pl.when(step == 1)
        def _init():
            out_ref[...] = y

        @pl.when(jnp.logical_and(step > 1, step < NFF))
        def _acc():
            out_ref[...] = out_ref[...] + y

        @pl.when(step == NFF)
        def _final():
            out_ref[...] = (out_ref[...] + y + b2_ref[...]) * coeff_ref[...]


@functools.partial(jax.jit, static_argnames=())
def kernel(inputs, Wg, bg, W1, b1, W2, b2):
    x = inputs.reshape(-1, M)

    out = pl.pallas_call(
        _fused_kernel,
        grid=(1 + NFF,),
        out_shape=jax.ShapeDtypeStruct((S, M), jnp.float32),
        in_specs=[
            pl.BlockSpec((S, M), lambda i: (0, 0)),
            pl.BlockSpec((M, E), lambda i: (0, 0)),
            pl.BlockSpec((1, E), lambda i: (0, 0)),
            pl.BlockSpec((M, FFCHUNK), lambda i: (0, jnp.maximum(i - 1, 0))),
            pl.BlockSpec((1, FFCHUNK), lambda i: (0, jnp.maximum(i - 1, 0))),
            pl.BlockSpec((FFCHUNK, M), lambda i: (jnp.maximum(i - 1, 0), 0)),
            pl.BlockSpec((1, M), lambda i: (0, 0)),
        ],
        out_specs=pl.BlockSpec((S, M), lambda i: (0, 0)),
        scratch_shapes=[
            pltpu.VMEM((S, 1), jnp.float32),
            pltpu.VMEM((S, M), jnp.bfloat16),
        ],
    )(x, Wg, bg.reshape(1, E), W1, b1.reshape(1, DFF), W2, b2.reshape(1, M))

    return out.reshape(inputs.shape)


# speedup vs baseline: 1.2219x; 1.2219x over previous
import functools

import jax
import jax.numpy as jnp
from jax.experimental import pallas as pl
from jax.experimental.pallas import tpu as pltpu

S = 2048
E = 8
M = 768
DFF = 3072
CUMCHUNK = 512
TOKBLK = 256


def _fused_kernel(
    x_ref, wg_ref, bg_ref, w1_ref, b1_ref, w2_ref, b2_ref, out_ref, coeff_ref
):
    step = pl.program_id(0)

    @pl.when(step == 0)
    def _gating():
        logits = jnp.dot(x_ref[...], wg_ref[...], preferred_element_type=jnp.float32)
        logits = logits + bg_ref[...]
        lt = jnp.transpose(logits)
        row = jax.lax.broadcasted_iota(jnp.int32, (E, S), 0)

        max1 = jnp.max(lt, axis=0, keepdims=True)
        idx1 = jnp.min(jnp.where(lt == max1, row, E), axis=0, keepdims=True)
        m1 = row == idx1

        neg_inf = jnp.float32(-jnp.inf)
        le1 = jnp.where(m1, neg_inf, lt)
        max2 = jnp.max(le1, axis=0, keepdims=True)
        idx2 = jnp.min(jnp.where(le1 == max2, row, E), axis=0, keepdims=True)
        m2 = row == idx2
        m1f = m1.astype(jnp.float32)
        m2f = m2.astype(jnp.float32)

        expx = jnp.exp(lt - max1)
        denom = jnp.sum(expx, axis=0, keepdims=True)
        g1 = jnp.sum(jnp.where(m1, expx, 0.0), axis=0, keepdims=True) / denom
        g2 = jnp.sum(jnp.where(m2, expx, 0.0), axis=0, keepdims=True) / denom

        cs2 = jnp.zeros((E, S), jnp.float32)
        for c in range(S // CUMCHUNK):
            t0 = c * CUMCHUNK
            rr = jax.lax.broadcasted_iota(jnp.int32, (CUMCHUNK, S), 0)
            cc = jax.lax.broadcasted_iota(jnp.int32, (CUMCHUNK, S), 1)
            ut = (rr + t0 <= cc).astype(jnp.float32)
            chunk = m2f[:, t0 : t0 + CUMCHUNK]
            cs2 = cs2 + jnp.dot(chunk, ut, preferred_element_type=jnp.float32)

        count1 = jnp.sum(m1f, axis=1, keepdims=True)
        loc2 = (
            jnp.sum(cs2 * m2f, axis=0, keepdims=True)
            - 1.0
            + jnp.sum(count1 * m2f, axis=0, keepdims=True)
        )
        valid2 = (loc2 < jnp.float32(S)).astype(jnp.float32)

        den = jnp.maximum(g1 + g2, jnp.float32(jnp.finfo(jnp.float32).eps))
        coeff_ref[...] = jnp.transpose((g1 + g2 * valid2) / den)

    @pl.when(step > 0)
    def _ffn():
        t0 = (step - 1) * TOKBLK
        xb = x_ref[pl.ds(t0, TOKBLK), :]
        h = jnp.dot(xb, w1_ref[...], preferred_element_type=jnp.float32)
        h = jnp.maximum(h + b1_ref[...], 0.0)
        y = jnp.dot(h, w2_ref[...], preferred_element_type=jnp.float32)
        out_ref[...] = (y + b2_ref[...]) * coeff_ref[pl.ds(t0, TOKBLK), :]


@functools.partial(jax.jit, static_argnames=())
def kernel(inputs, Wg, bg, W1, b1, W2, b2):
    x = inputs.reshape(-1, M)

    out = pl.pallas_call(
        _fused_kernel,
        grid=(1 + S // TOKBLK,),
        out_shape=jax.ShapeDtypeStruct((S, M), jnp.float32),
        in_specs=[
            pl.BlockSpec((S, M), lambda i: (0, 0)),
            pl.BlockSpec((M, E), lambda i: (0, 0)),
            pl.BlockSpec((1, E), lambda i: (0, 0)),
            pl.BlockSpec((M, DFF), lambda i: (0, 0)),
            pl.BlockSpec((1, DFF), lambda i: (0, 0)),
            pl.BlockSpec((DFF, M), lambda i: (0, 0)),
            pl.BlockSpec((1, M), lambda i: (0, 0)),
        ],
        out_specs=pl.BlockSpec(
            (TOKBLK, M), lambda i: (jnp.maximum(i - 1, 0), 0)
        ),
        scratch_shapes=[
            pltpu.VMEM((S, 1), jnp.float32),
        ],
    )(x, Wg, bg.reshape(1, E), W1, b1.reshape(1, DFF), W2, b2.reshape(1, M))

    return out.reshape(inputs.shape)


# speedup vs baseline: 1.2719x; 1.0409x over previous
import functools

import jax
import jax.numpy as jnp
from jax.experimental import pallas as pl
from jax.experimental.pallas import tpu as pltpu

S = 2048
E = 8
M = 768
DFF = 3072
CUMCHUNK = 512
TOKBLK = 256


def _fused_kernel(
    x_ref, wg_ref, bg_ref, w1_ref, b1_ref, w2_ref, b2_ref, out_ref,
    coeff_ref, xb_ref, w1v_ref, w2v_ref, w1b_ref, w2b_ref, sem1, sem2,
):
    step = pl.program_id(0)

    @pl.when(step == 0)
    def _start_weight_copies():
        pltpu.make_async_copy(w1_ref, w1v_ref, sem1).start()
        pltpu.make_async_copy(w2_ref, w2v_ref, sem2).start()

    @pl.when(step == 0)
    def _gating():
        xb_ref[...] = x_ref[...].astype(jnp.bfloat16)
        logits = jnp.dot(x_ref[...], wg_ref[...], preferred_element_type=jnp.float32)
        logits = logits + bg_ref[...]
        lt = jnp.transpose(logits)
        row = jax.lax.broadcasted_iota(jnp.int32, (E, S), 0)

        max1 = jnp.max(lt, axis=0, keepdims=True)
        idx1 = jnp.min(jnp.where(lt == max1, row, E), axis=0, keepdims=True)
        m1 = row == idx1

        neg_inf = jnp.float32(-jnp.inf)
        le1 = jnp.where(m1, neg_inf, lt)
        max2 = jnp.max(le1, axis=0, keepdims=True)
        idx2 = jnp.min(jnp.where(le1 == max2, row, E), axis=0, keepdims=True)
        m2 = row == idx2
        m1f = m1.astype(jnp.float32)
        m2f = m2.astype(jnp.float32)

        expx = jnp.exp(lt - max1)
        denom = jnp.sum(expx, axis=0, keepdims=True)
        g1 = jnp.sum(jnp.where(m1, expx, 0.0), axis=0, keepdims=True) / denom
        g2 = jnp.sum(jnp.where(m2, expx, 0.0), axis=0, keepdims=True) / denom

        cs2 = jnp.zeros((E, S), jnp.float32)
        for c in range(S // CUMCHUNK):
            t0 = c * CUMCHUNK
            rr = jax.lax.broadcasted_iota(jnp.int32, (CUMCHUNK, S), 0)
            cc = jax.lax.broadcasted_iota(jnp.int32, (CUMCHUNK, S), 1)
            ut = (rr + t0 <= cc).astype(jnp.float32)
            chunk = m2f[:, t0 : t0 + CUMCHUNK]
            cs2 = cs2 + jnp.dot(chunk, ut, preferred_element_type=jnp.float32)

        count1 = jnp.sum(m1f, axis=1, keepdims=True)
        loc2 = (
            jnp.sum(cs2 * m2f, axis=0, keepdims=True)
            - 1.0
            + jnp.sum(count1 * m2f, axis=0, keepdims=True)
        )
        valid2 = (loc2 < jnp.float32(S)).astype(jnp.float32)

        den = jnp.maximum(g1 + g2, jnp.float32(jnp.finfo(jnp.float32).eps))
        coeff_ref[...] = jnp.transpose((g1 + g2 * valid2) / den)

    @pl.when(step == 1)
    def _wait_and_cast_weights():
        pltpu.make_async_copy(w1_ref, w1v_ref, sem1).wait()
        pltpu.make_async_copy(w2_ref, w2v_ref, sem2).wait()
        w1b_ref[...] = w1v_ref[...].astype(jnp.bfloat16)
        w2b_ref[...] = w2v_ref[...].astype(jnp.bfloat16)

    @pl.when(step > 0)
    def _ffn():
        t0 = (step - 1) * TOKBLK
        xb = xb_ref[pl.ds(t0, TOKBLK), :]
        h = jnp.dot(xb, w1b_ref[...], preferred_element_type=jnp.float32)
        h = jnp.maximum(h + b1_ref[...], 0.0).astype(jnp.bfloat16)
        y = jnp.dot(h, w2b_ref[...], preferred_element_type=jnp.float32)
        out_ref[...] = (y + b2_ref[...]) * coeff_ref[pl.ds(t0, TOKBLK), :]


@functools.partial(jax.jit, static_argnames=())
def kernel(inputs, Wg, bg, W1, b1, W2, b2):
    x = inputs.reshape(-1, M)

    out = pl.pallas_call(
        _fused_kernel,
        grid=(1 + S // TOKBLK,),
        out_shape=jax.ShapeDtypeStruct((S, M), jnp.float32),
        in_specs=[
            pl.BlockSpec((S, M), lambda i: (0, 0)),
            pl.BlockSpec((M, E), lambda i: (0, 0)),
            pl.BlockSpec((1, E), lambda i: (0, 0)),
            pl.BlockSpec(memory_space=pl.ANY),
            pl.BlockSpec((1, DFF), lambda i: (0, 0)),
            pl.BlockSpec(memory_space=pl.ANY),
            pl.BlockSpec((1, M), lambda i: (0, 0)),
        ],
        out_specs=pl.BlockSpec(
            (TOKBLK, M), lambda i: (jnp.maximum(i - 1, 0), 0)
        ),
        scratch_shapes=[
            pltpu.VMEM((S, 1), jnp.float32),
            pltpu.VMEM((S, M), jnp.bfloat16),
            pltpu.VMEM((M, DFF), jnp.float32),
            pltpu.VMEM((DFF, M), jnp.float32),
            pltpu.VMEM((M, DFF), jnp.bfloat16),
            pltpu.VMEM((DFF, M), jnp.bfloat16),
            pltpu.SemaphoreType.DMA,
            pltpu.SemaphoreType.DMA,
        ],
    )(x, Wg, bg.reshape(1, E), W1, b1.reshape(1, DFF), W2, b2.reshape(1, M))

    return out.reshape(inputs.shape)


# speedup vs baseline: 1.3406x; 1.0541x over previous
import functools

import jax
import jax.numpy as jnp
from jax.experimental import pallas as pl
from jax.experimental.pallas import tpu as pltpu

S = 2048
M = 768
DFF = 3072
TOKBLK = 256


def _ffn_kernel(
    x_ref, w1_ref, b1_ref, w2_ref, b2_ref, out_ref,
    xb_ref, w1v_ref, w2v_ref, w1b_ref, w2b_ref, sem1, sem2,
):
    step = pl.program_id(0)

    @pl.when(step == 0)
    def _start():
        pltpu.make_async_copy(w1_ref, w1v_ref, sem1).start()
        pltpu.make_async_copy(w2_ref, w2v_ref, sem2).start()
        xb_ref[...] = x_ref[...].astype(jnp.bfloat16)

    @pl.when(step == 1)
    def _wait_and_cast_weights():
        pltpu.make_async_copy(w1_ref, w1v_ref, sem1).wait()
        pltpu.make_async_copy(w2_ref, w2v_ref, sem2).wait()
        w1b_ref[...] = w1v_ref[...].astype(jnp.bfloat16)
        w2b_ref[...] = w2v_ref[...].astype(jnp.bfloat16)

    @pl.when(step > 0)
    def _ffn():
        t0 = (step - 1) * TOKBLK
        xb = xb_ref[pl.ds(t0, TOKBLK), :]
        h = jnp.dot(xb, w1b_ref[...], preferred_element_type=jnp.float32)
        h = jnp.maximum(h + b1_ref[...], 0.0).astype(jnp.bfloat16)
        y = jnp.dot(h, w2b_ref[...], preferred_element_type=jnp.float32)
        out_ref[...] = y + b2_ref[...]


@functools.partial(jax.jit, static_argnames=())
def kernel(inputs, Wg, bg, W1, b1, W2, b2):
    x = inputs.reshape(-1, M)

    out = pl.pallas_call(
        _ffn_kernel,
        grid=(1 + S // TOKBLK,),
        out_shape=jax.ShapeDtypeStruct((S, M), jnp.float32),
        in_specs=[
            pl.BlockSpec((S, M), lambda i: (0, 0)),
            pl.BlockSpec(memory_space=pl.ANY),
            pl.BlockSpec((1, DFF), lambda i: (0, 0)),
            pl.BlockSpec(memory_space=pl.ANY),
            pl.BlockSpec((1, M), lambda i: (0, 0)),
        ],
        out_specs=pl.BlockSpec(
            (TOKBLK, M), lambda i: (jnp.maximum(i - 1, 0), 0)
        ),
        scratch_shapes=[
            pltpu.VMEM((S, M), jnp.bfloat16),
            pltpu.VMEM((M, DFF), jnp.float32),
            pltpu.VMEM((DFF, M), jnp.float32),
            pltpu.VMEM((M, DFF), jnp.bfloat16),
            pltpu.VMEM((DFF, M), jnp.bfloat16),
            pltpu.SemaphoreType.DMA,
            pltpu.SemaphoreType.DMA,
        ],
    )(x, W1, b1.reshape(1, DFF), W2, b2.reshape(1, M))

    return out.reshape(inputs.shape)


# speedup vs baseline: 1.5487x; 1.1552x over previous
import functools

import jax
import jax.numpy as jnp
from jax.experimental import pallas as pl
from jax.experimental.pallas import tpu as pltpu

S = 2048
M = 768
DFF = 3072
TOKBLK = 256
NTOK = S // TOKBLK
NW1 = 4
DFFBLK = DFF // NW1


def _ffn_kernel(
    x_ref, w1_ref, b1_ref, w2_ref, b2_ref, out_ref,
    xb_ref, h_ref, w1v_ref, w2v_ref, w1b_ref, w2b_ref, sem1, sem2,
):
    step = pl.program_id(0)

    @pl.when(step == 0)
    def _start():
        for j in range(NW1):
            pltpu.make_async_copy(
                w1_ref.at[:, pl.ds(j * DFFBLK, DFFBLK)],
                w1v_ref.at[:, pl.ds(j * DFFBLK, DFFBLK)],
                sem1.at[j],
            ).start()
        pltpu.make_async_copy(w2_ref, w2v_ref, sem2).start()
        xb_ref[...] = x_ref[...].astype(jnp.bfloat16)

    @pl.when(step < NW1)
    def _h_phase():
        off = pl.multiple_of(step * DFFBLK, DFFBLK)
        pltpu.make_async_copy(
            w1_ref.at[:, pl.ds(off, DFFBLK)],
            w1v_ref.at[:, pl.ds(off, DFFBLK)],
            sem1.at[step],
        ).wait()
        w1b_ref[...] = w1v_ref[:, pl.ds(off, DFFBLK)].astype(jnp.bfloat16)
        h = jnp.dot(xb_ref[...], w1b_ref[...], preferred_element_type=jnp.float32)
        h = jnp.maximum(h + b1_ref[:, pl.ds(off, DFFBLK)], 0.0)
        h_ref[:, pl.ds(off, DFFBLK)] = h.astype(jnp.bfloat16)

    @pl.when(step == NW1)
    def _wait_w2():
        pltpu.make_async_copy(w2_ref, w2v_ref, sem2).wait()
        w2b_ref[...] = w2v_ref[...].astype(jnp.bfloat16)

    @pl.when(step >= NW1)
    def _y_phase():
        t0 = pl.multiple_of((step - NW1) * TOKBLK, TOKBLK)
        h = h_ref[pl.ds(t0, TOKBLK), :]
        y = jnp.dot(h, w2b_ref[...], preferred_element_type=jnp.float32)
        out_ref[...] = y + b2_ref[...]


@functools.partial(jax.jit, static_argnames=())
def kernel(inputs, Wg, bg, W1, b1, W2, b2):
    x = inputs.reshape(-1, M)

    out = pl.pallas_call(
        _ffn_kernel,
        grid=(NW1 + NTOK,),
        out_shape=jax.ShapeDtypeStruct((S, M), jnp.float32),
        in_specs=[
            pl.BlockSpec((S, M), lambda i: (0, 0)),
            pl.BlockSpec(memory_space=pl.ANY),
            pl.BlockSpec((1, DFF), lambda i: (0, 0)),
            pl.BlockSpec(memory_space=pl.ANY),
            pl.BlockSpec((1, M), lambda i: (0, 0)),
        ],
        out_specs=pl.BlockSpec(
            (TOKBLK, M), lambda i: (jnp.maximum(i - NW1, 0), 0)
        ),
        scratch_shapes=[
            pltpu.VMEM((S, M), jnp.bfloat16),
            pltpu.VMEM((S, DFF), jnp.bfloat16),
            pltpu.VMEM((M, DFF), jnp.float32),
            pltpu.VMEM((DFF, M), jnp.float32),
            pltpu.VMEM((M, DFFBLK), jnp.bfloat16),
            pltpu.VMEM((DFF, M), jnp.bfloat16),
            pltpu.SemaphoreType.DMA((NW1,)),
            pltpu.SemaphoreType.DMA,
        ],
        compiler_params=pltpu.CompilerParams(
            vmem_limit_bytes=110 * 1024 * 1024,
        ),
    )(x, W1, b1.reshape(1, DFF), W2, b2.reshape(1, M))

    return out.reshape(inputs.shape)


# speedup vs baseline: 1.5572x; 1.0055x over previous
import functools

import jax
import jax.numpy as jnp
from jax.experimental import pallas as pl
from jax.experimental.pallas import tpu as pltpu

S = 2048
M = 768
DFF = 3072
TOKBLK = 256
NTOK = S // TOKBLK
NW1 = 4
DFFBLK = DFF // NW1


def _ffn_kernel(
    x_ref, w1_ref, b1_ref, w2_ref, b2_ref, out_ref,
    xb_ref, h_ref, w1v_ref, w2v_ref, w1b_ref, w2b_ref, sem1, sem2,
):
    step = pl.program_id(0)

    @pl.when(step == 0)
    def _start():
        for j in range(NW1):
            pltpu.make_async_copy(
                w1_ref.at[:, pl.ds(j * DFFBLK, DFFBLK)],
                w1v_ref.at[:, pl.ds(j * DFFBLK, DFFBLK)],
                sem1.at[j],
            ).start()
            pltpu.make_async_copy(
                w2_ref.at[pl.ds(j * DFFBLK, DFFBLK), :],
                w2v_ref.at[pl.ds(j * DFFBLK, DFFBLK), :],
                sem2.at[j],
            ).start()
        xb_ref[...] = x_ref[...].astype(jnp.bfloat16)

    @pl.when(step < NW1)
    def _h_phase():
        off = pl.multiple_of(step * DFFBLK, DFFBLK)
        pltpu.make_async_copy(
            w1_ref.at[:, pl.ds(off, DFFBLK)],
            w1v_ref.at[:, pl.ds(off, DFFBLK)],
            sem1.at[step],
        ).wait()
        w1b_ref[...] = w1v_ref[:, pl.ds(off, DFFBLK)].astype(jnp.bfloat16)
        h = jnp.dot(xb_ref[...], w1b_ref[...], preferred_element_type=jnp.float32)
        h = jnp.maximum(h + b1_ref[:, pl.ds(off, DFFBLK)], 0.0)
        h_ref[:, pl.ds(off, DFFBLK)] = h.astype(jnp.bfloat16)

    @pl.when((step >= 1) & (step <= NW1))
    def _w2_chunk():
        j = pl.multiple_of((step - 1) * DFFBLK, DFFBLK)
        pltpu.make_async_copy(
            w2_ref.at[pl.ds(j, DFFBLK), :],
            w2v_ref.at[pl.ds(j, DFFBLK), :],
            sem2.at[step - 1],
        ).wait()
        w2b_ref[pl.ds(j, DFFBLK), :] = w2v_ref[pl.ds(j, DFFBLK), :].astype(
            jnp.bfloat16
        )

    @pl.when(step >= NW1)
    def _y_phase():
        t0 = pl.multiple_of((step - NW1) * TOKBLK, TOKBLK)
        h = h_ref[pl.ds(t0, TOKBLK), :]
        y = jnp.dot(h, w2b_ref[...], preferred_element_type=jnp.float32)
        out_ref[...] = y + b2_ref[...]


@functools.partial(jax.jit, static_argnames=())
def kernel(inputs, Wg, bg, W1, b1, W2, b2):
    x = inputs.reshape(-1, M)

    out = pl.pallas_call(
        _ffn_kernel,
        grid=(NW1 + NTOK,),
        out_shape=jax.ShapeDtypeStruct((S, M), jnp.float32),
        in_specs=[
            pl.BlockSpec((S, M), lambda i: (0, 0)),
            pl.BlockSpec(memory_space=pl.ANY),
            pl.BlockSpec((1, DFF), lambda i: (0, 0)),
            pl.BlockSpec(memory_space=pl.ANY),
            pl.BlockSpec((1, M), lambda i: (0, 0)),
        ],
        out_specs=pl.BlockSpec(
            (TOKBLK, M), lambda i: (jnp.maximum(i - NW1, 0), 0)
        ),
        scratch_shapes=[
            pltpu.VMEM((S, M), jnp.bfloat16),
            pltpu.VMEM((S, DFF), jnp.bfloat16),
            pltpu.VMEM((M, DFF), jnp.float32),
            pltpu.VMEM((DFF, M), jnp.float32),
            pltpu.VMEM((M, DFFBLK), jnp.bfloat16),
            pltpu.VMEM((DFF, M), jnp.bfloat16),
            pltpu.SemaphoreType.DMA((NW1,)),
            pltpu.SemaphoreType.DMA((NW1,)),
        ],
        compiler_params=pltpu.CompilerParams(
            vmem_limit_bytes=110 * 1024 * 1024,
        ),
    )(x, W1, b1.reshape(1, DFF), W2, b2.reshape(1, M))

    return out.reshape(inputs.shape)
